# baseline (speedup 1.0000x reference)
import functools

import jax
import jax.numpy as jnp
from jax import lax
from jax.experimental import pallas as pl
from jax.experimental.pallas import tpu as pltpu
from jax.experimental.pallas import tpu_sc as plsc

B, C, H, W = 64, 768, 14, 14
HW = H * W
NCLS = 1000
LOSS_OFFSET = C * NCLS * 0.2

HBLK = 14
NBLK = HW // HBLK

NTILES = 16
NWORDS = C * NCLS
WPT = NWORDS // NTILES
VCHUNKS = WPT // 16


def _sc_loss(maskf_hbm, parts_hbm, buf_v, acc_v):
    cid = lax.axis_index("c")
    sid = lax.axis_index("s")

    @pl.when(cid == 0)
    def _():
        pltpu.sync_copy(maskf_hbm.at[pl.ds(sid * WPT, WPT)], buf_v)

        def body(i, acc):
            return acc + jnp.abs(buf_v[pl.ds(i * 16, 16)])

        acc = lax.fori_loop(0, VCHUNKS, body, jnp.zeros((16,), jnp.float32))
        acc_v[...] = acc
        pltpu.sync_copy(acc_v, parts_hbm.at[sid])


_sc_loss_call = functools.partial(
    pl.kernel,
    mesh=plsc.VectorSubcoreMesh(core_axis_name="c", subcore_axis_name="s"),
    out_type=jax.ShapeDtypeStruct((NTILES, 16), jnp.float32),
    scratch_types=[
        pltpu.VMEM((WPT,), jnp.float32),
        pltpu.VMEM((16,), jnp.float32),
    ],
)(_sc_loss)


def _mul_kernel(labels_ref, mask_t_ref, x_ref, out_ref, scales_ref):
    @pl.when(pl.program_id(0) == 0)
    def _():
        labels_v = labels_ref[...]
        iota = jax.lax.broadcasted_iota(jnp.int32, (B, NCLS), 1)
        onehot = (iota == labels_v[:, None]).astype(jnp.float32)
        scales_ref[...] = jax.lax.dot_general(
            onehot, mask_t_ref[...],
            dimension_numbers=(((1,), (0,)), ((), ())),
            preferred_element_type=jnp.float32,
        )

    out_ref[...] = x_ref[...] * scales_ref[...][None, :, :]


def kernel(x, labels, mask):
    xt = jnp.transpose(x, (2, 3, 0, 1)).reshape(HW, B, C)
    mask_t = mask.T
    mask_flat = mask_t.reshape(NWORDS)

    parts = _sc_loss_call(mask_flat)

    out_t = pl.pallas_call(
        _mul_kernel,
        grid=(NBLK,),
        in_specs=[
            pl.BlockSpec(memory_space=pltpu.VMEM),
            pl.BlockSpec((NCLS, C), lambda i: (0, 0)),
            pl.BlockSpec((HBLK, B, C), lambda i: (i, 0, 0)),
        ],
        out_specs=pl.BlockSpec((HBLK, B, C), lambda i: (i, 0, 0)),
        out_shape=jax.ShapeDtypeStruct((HW, B, C), x.dtype),
        scratch_shapes=[pltpu.VMEM((B, C), jnp.float32)],
    )(labels, mask_t, xt)
    out = jnp.transpose(out_t.reshape(H, W, B, C), (2, 3, 0, 1))
    loss = jnp.maximum(jnp.sum(parts) - LOSS_OFFSET, 0.0)
    return out, loss
